# Initial kernel scaffold; baseline (speedup 1.0000x reference)
import jax
import jax.numpy as jnp
from jax.experimental import pallas as pl


def kernel(x, grid):
    raise NotImplementedError("write your pallas kernel here")


# speedup vs baseline: 91.6501x; 91.6501x over previous
import functools

import jax
import jax.numpy as jnp
from jax import lax
from jax.experimental import pallas as pl
from jax.experimental.pallas import tpu as pltpu
from jax.experimental.pallas import tpu_sc as plsc

N = 1048576
GRID = 128
NW = 32
NPER = N // NW
C = 2048
NCHUNK = NPER // C
J = C // 128
L = 16

OFFS = (0, 1, 128, 129, 16384, 16385, 16512, 16513)


def _body(x_hbm, g_hbm, o_hbm, xbuf, fbuf, idxbuf, vbuf, obuf, sem):
    wid = lax.axis_index("s") * 2 + lax.axis_index("c")
    base_pt = wid * NPER

    iota = lax.iota(jnp.int32, L)
    scale = jnp.full((L,), 125.0, jnp.float32)
    one = jnp.full((L,), 1.0, jnp.float32)

    iota3 = iota * 3

    def chunk_body(t, carry):
        off = base_pt + t * C
        pltpu.sync_copy(x_hbm.at[pl.ds(off * 3, C * 3)], xbuf)

        def row_fire(j, carry):
            for g in range(8):
                p = j * 128 + g * L
                pidx = iota3 + (3 * p)
                x0 = plsc.load_gather(xbuf, [pidx])
                x1 = plsc.load_gather(xbuf, [pidx + 1])
                x2 = plsc.load_gather(xbuf, [pidx + 2])
                t0 = x0 * scale
                t1 = x1 * scale
                t2 = x2 * scale
                i0 = jnp.clip(t0.astype(jnp.int32), 0, GRID - 2)
                i1 = jnp.clip(t1.astype(jnp.int32), 0, GRID - 2)
                i2 = jnp.clip(t2.astype(jnp.int32), 0, GRID - 2)
                fbuf[0, pl.ds(p, L)] = t0 - i0.astype(jnp.float32)
                fbuf[1, pl.ds(p, L)] = t1 - i1.astype(jnp.float32)
                fbuf[2, pl.ds(p, L)] = t2 - i2.astype(jnp.float32)
                flat = (
                    lax.shift_left(i0, 14) + lax.shift_left(i1, 7) + i2
                )
                for c8 in range(8):
                    idxbuf[c8, j, pl.ds(g * L, L)] = flat + OFFS[c8]
            for c8 in range(8):
                pltpu.async_copy(
                    g_hbm.at[idxbuf.at[c8, j]],
                    vbuf.at[c8, pl.ds(j * 128, 128)],
                    sem,
                )
            return carry

        lax.fori_loop(0, J, row_fire, 0, unroll=False)

        def row_drain(j, carry):
            for c8 in range(8):
                pltpu.make_async_copy(
                    g_hbm.at[idxbuf.at[c8, j]],
                    vbuf.at[c8, pl.ds(j * 128, 128)],
                    sem,
                ).wait()
            return carry

        lax.fori_loop(0, J, row_drain, 0, unroll=False)

        def row_acc(j, carry):
            for g in range(8):
                p = j * 128 + g * L
                f0 = fbuf[0, pl.ds(p, L)]
                f1 = fbuf[1, pl.ds(p, L)]
                f2 = fbuf[2, pl.ds(p, L)]
                g0 = one - f0
                g1 = one - f1
                g2 = one - f2
                acc = None
                for c8 in range(8):
                    wx = f0 if (c8 & 4) else g0
                    wy = f1 if (c8 & 2) else g1
                    wz = f2 if (c8 & 1) else g2
                    w = wx * wy * wz
                    v = vbuf[c8, pl.ds(p, L)]
                    term = w * v
                    acc = term if acc is None else acc + term
                obuf[pl.ds(p, L)] = acc
            return carry

        lax.fori_loop(0, J, row_acc, 0, unroll=False)

        pltpu.sync_copy(obuf, o_hbm.at[pl.ds(off, C)])
        return carry

    lax.fori_loop(0, NCHUNK, chunk_body, 0, unroll=False)


@jax.jit
def _run(x, gflat):
    mesh = plsc.VectorSubcoreMesh(core_axis_name="c", subcore_axis_name="s")
    kern = pl.kernel(
        _body,
        out_type=jax.ShapeDtypeStruct((N,), jnp.float32),
        mesh=mesh,
        scratch_types=[
            pltpu.VMEM((C * 3,), jnp.float32),
            pltpu.VMEM((3, C), jnp.float32),
            pltpu.VMEM((8, J, 128), jnp.int32),
            pltpu.VMEM((8, C), jnp.float32),
            pltpu.VMEM((C,), jnp.float32),
            pltpu.SemaphoreType.DMA,
        ],
        compiler_params=pltpu.CompilerParams(needs_layout_passes=False),
    )
    return kern(x, gflat)


def kernel(x, grid):
    return _run(x.reshape(-1), grid.reshape(-1)).reshape(N, 1)
